# Initial kernel scaffold; baseline (speedup 1.0000x reference)
import functools

import jax
import jax.numpy as jnp
from jax import lax
from jax.experimental import pallas as pl
from jax.experimental.pallas import tpu as pltpu
from jax.experimental.pallas import tpu_sc as plsc

N = 10000
D = 128
NC = 2
NS = 16
NW = NC * NS
CHUNK = 128
N_TAB = 10016
ROWS_PER_TILE = N_TAB // NS
TRASH = N


def _sc_aggregate(h, src3, dst3, zrows):
  nchunks = src3.shape[1]
  mesh = plsc.VectorSubcoreMesh(core_axis_name="c", subcore_axis_name="s")

  @functools.partial(
      pl.kernel,
      out_type=jax.ShapeDtypeStruct((NC, N_TAB, D), jnp.float32),
      mesh=mesh,
      scratch_types=[
          pltpu.VMEM((nchunks, CHUNK), jnp.int32),
          pltpu.VMEM((nchunks, CHUNK), jnp.int32),
          pltpu.VMEM((CHUNK, D), jnp.float32),
          pltpu.VMEM_SHARED((N_TAB, D), jnp.float32),
          pltpu.SemaphoreType.DMA,
      ],
  )
  def k(h_hbm, src_hbm, dst_hbm, z_hbm, out_hbm, src_v, dst_v, rows_v, agg,
        sem):
    c = lax.axis_index("c")
    s = lax.axis_index("s")
    wid = s * NC + c
    pltpu.sync_copy(src_hbm.at[wid], src_v)
    pltpu.sync_copy(dst_hbm.at[wid], dst_v)
    pltpu.sync_copy(z_hbm, agg.at[pl.ds(s * ROWS_PER_TILE, ROWS_PER_TILE)])
    plsc.subcore_barrier()

    @pl.loop(0, nchunks)
    def _(j):
      pltpu.async_copy(h_hbm.at[src_v.at[j]], rows_v, sem).wait()
      pltpu.sync_copy(rows_v, agg.at[dst_v.at[j]], add=True)

    plsc.subcore_barrier()
    pltpu.sync_copy(agg.at[pl.ds(s * ROWS_PER_TILE, ROWS_PER_TILE)],
                    out_hbm.at[c, pl.ds(s * ROWS_PER_TILE, ROWS_PER_TILE)])

  return k(h, src3, dst3, zrows)


def _mlp_body(h_ref, p0_ref, p1_ref, w1_ref, b1_ref, w2_ref, b2_ref, out_ref):
  u = h_ref[...] + p0_ref[...] + p1_ref[...]
  t = jnp.dot(u, w1_ref[...], preferred_element_type=jnp.float32) + b1_ref[...]
  t = jax.nn.gelu(t)
  v = jnp.dot(t, w2_ref[...], preferred_element_type=jnp.float32) + b2_ref[...]
  out_ref[...] = jax.nn.gelu(v)


def _mlp_head_body(h_ref, p0_ref, p1_ref, w1_ref, b1_ref, w2_ref, b2_ref,
                   wp_ref, bp_ref, out_ref):
  u = h_ref[...] + p0_ref[...] + p1_ref[...]
  t = jnp.dot(u, w1_ref[...], preferred_element_type=jnp.float32) + b1_ref[...]
  t = jax.nn.gelu(t)
  v = jnp.dot(t, w2_ref[...], preferred_element_type=jnp.float32) + b2_ref[...]
  g = jax.nn.gelu(v)
  out_ref[...] = (
      jnp.dot(g, wp_ref[...], preferred_element_type=jnp.float32) + bp_ref[...])


_ROW_BLK = 1000


def _row_spec():
  return pl.BlockSpec((_ROW_BLK, D), lambda i: (i, 0))


def _full_spec(shape):
  return pl.BlockSpec(shape, lambda i: tuple(0 for _ in shape))


def _tc_mlp(h, p0, p1, w1, b1, w2, b2):
  grid = (N // _ROW_BLK,)
  return pl.pallas_call(
      _mlp_body,
      grid=grid,
      in_specs=[_row_spec(), _row_spec(), _row_spec(),
                _full_spec((D, D)), _full_spec((1, D)),
                _full_spec((D, D)), _full_spec((1, D))],
      out_specs=_row_spec(),
      out_shape=jax.ShapeDtypeStruct((N, D), jnp.float32),
  )(h, p0, p1, w1, b1.reshape(1, D), w2, b2.reshape(1, D))


def _tc_mlp_head(h, p0, p1, w1, b1, w2, b2, wp, bp):
  grid = (N // _ROW_BLK,)
  return pl.pallas_call(
      _mlp_head_body,
      grid=grid,
      in_specs=[_row_spec(), _row_spec(), _row_spec(),
                _full_spec((D, D)), _full_spec((1, D)),
                _full_spec((D, D)), _full_spec((1, D)),
                _full_spec((D, D)), _full_spec((1, D))],
      out_specs=_row_spec(),
      out_shape=jax.ShapeDtypeStruct((N, D), jnp.float32),
  )(h, p0, p1, w1, b1.reshape(1, D), w2, b2.reshape(1, D), wp,
    bp.reshape(1, D))


def kernel(x, edge_index, W1a, b1a, W2a, b2a, W1b, b1b, W2b, b2b, Wp, bp):
  src = edge_index[0]
  dst = edge_index[1]
  e = src.shape[0]
  nchunks = -(-e // (NW * CHUNK))
  e_pad = NW * CHUNK * nchunks
  src3 = jnp.concatenate(
      [src, jnp.zeros((e_pad - e,), jnp.int32)]).reshape(NW, nchunks, CHUNK)
  dst3 = jnp.concatenate(
      [dst, jnp.full((e_pad - e,), TRASH, jnp.int32)]).reshape(
          NW, nchunks, CHUNK)
  zrows = jnp.zeros((ROWS_PER_TILE, D), jnp.float32)

  agg_a = _sc_aggregate(x, src3, dst3, zrows)
  h1 = _tc_mlp(x, agg_a[0, :N], agg_a[1, :N], W1a, b1a, W2a, b2a)
  agg_b = _sc_aggregate(h1, src3, dst3, zrows)
  return _tc_mlp_head(h1, agg_b[0, :N], agg_b[1, :N], W1b, b1b, W2b, b2b,
                      Wp, bp)


# speedup vs baseline: 4.0749x; 4.0749x over previous
import functools

import jax
import jax.numpy as jnp
from jax import lax
from jax.experimental import pallas as pl
from jax.experimental.pallas import tpu as pltpu
from jax.experimental.pallas import tpu_sc as plsc

N = 10000
D = 128
NC = 2
NS = 16
NW = NC * NS
CHUNK = 128
N_TAB = 10112
ROWS_PER_TILE = N_TAB // NS
TRASH = N


def _sc_aggregate(h, src3, dst3, zrows):
  nchunks = src3.shape[1]
  mesh = plsc.VectorSubcoreMesh(core_axis_name="c", subcore_axis_name="s")

  @functools.partial(
      pl.kernel,
      out_type=jax.ShapeDtypeStruct((NC, N_TAB, D), jnp.float32),
      mesh=mesh,
      scratch_types=[
          pltpu.VMEM((nchunks, CHUNK), jnp.int32),
          pltpu.VMEM((nchunks, CHUNK), jnp.int32),
          pltpu.VMEM((CHUNK, D), jnp.float32),
          pltpu.VMEM_SHARED((N_TAB, D), jnp.float32),
          pltpu.SemaphoreType.DMA,
      ],
  )
  def k(h_hbm, src_hbm, dst_hbm, z_hbm, out_hbm, src_v, dst_v, rows_v, agg,
        sem):
    c = lax.axis_index("c")
    s = lax.axis_index("s")
    wid = s * NC + c
    pltpu.sync_copy(src_hbm.at[wid], src_v)
    pltpu.sync_copy(dst_hbm.at[wid], dst_v)
    pltpu.sync_copy(z_hbm, agg.at[pl.ds(s * ROWS_PER_TILE, ROWS_PER_TILE)])
    plsc.subcore_barrier()

    @pl.loop(0, nchunks)
    def _(j):
      pltpu.async_copy(h_hbm.at[src_v.at[j]], rows_v, sem).wait()
      pltpu.sync_copy(rows_v, agg.at[dst_v.at[j]], add=True)

    plsc.subcore_barrier()
    pltpu.sync_copy(agg.at[pl.ds(s * ROWS_PER_TILE, ROWS_PER_TILE)],
                    out_hbm.at[c, pl.ds(s * ROWS_PER_TILE, ROWS_PER_TILE)])

  return k(h, src3, dst3, zrows)


def _mlp_body(h_ref, p0_ref, p1_ref, w1_ref, b1_ref, w2_ref, b2_ref, out_ref):
  u = h_ref[...] + p0_ref[...] + p1_ref[...]
  t = jnp.dot(u, w1_ref[...], preferred_element_type=jnp.float32) + b1_ref[...]
  t = jax.nn.gelu(t)
  v = jnp.dot(t, w2_ref[...], preferred_element_type=jnp.float32) + b2_ref[...]
  out_ref[...] = jax.nn.gelu(v)


def _mlp_head_body(h_ref, p0_ref, p1_ref, w1_ref, b1_ref, w2_ref, b2_ref,
                   wp_ref, bp_ref, out_ref):
  u = h_ref[...] + p0_ref[...] + p1_ref[...]
  t = jnp.dot(u, w1_ref[...], preferred_element_type=jnp.float32) + b1_ref[...]
  t = jax.nn.gelu(t)
  v = jnp.dot(t, w2_ref[...], preferred_element_type=jnp.float32) + b2_ref[...]
  g = jax.nn.gelu(v)
  out_ref[...] = (
      jnp.dot(g, wp_ref[...], preferred_element_type=jnp.float32) + bp_ref[...])


_ROW_BLK = 1000


def _row_spec():
  return pl.BlockSpec((_ROW_BLK, D), lambda i: (i, 0))


def _full_spec(shape):
  return pl.BlockSpec(shape, lambda i: tuple(0 for _ in shape))


def _tc_mlp(h, p0, p1, w1, b1, w2, b2):
  grid = (N // _ROW_BLK,)
  return pl.pallas_call(
      _mlp_body,
      grid=grid,
      in_specs=[_row_spec(), _row_spec(), _row_spec(),
                _full_spec((D, D)), _full_spec((1, D)),
                _full_spec((D, D)), _full_spec((1, D))],
      out_specs=_row_spec(),
      out_shape=jax.ShapeDtypeStruct((N, D), jnp.float32),
  )(h, p0, p1, w1, b1.reshape(1, D), w2, b2.reshape(1, D))


def _tc_mlp_head(h, p0, p1, w1, b1, w2, b2, wp, bp):
  grid = (N // _ROW_BLK,)
  return pl.pallas_call(
      _mlp_head_body,
      grid=grid,
      in_specs=[_row_spec(), _row_spec(), _row_spec(),
                _full_spec((D, D)), _full_spec((1, D)),
                _full_spec((D, D)), _full_spec((1, D)),
                _full_spec((D, D)), _full_spec((1, D))],
      out_specs=_row_spec(),
      out_shape=jax.ShapeDtypeStruct((N, D), jnp.float32),
  )(h, p0, p1, w1, b1.reshape(1, D), w2, b2.reshape(1, D), wp,
    bp.reshape(1, D))


def kernel(x, edge_index, W1a, b1a, W2a, b2a, W1b, b1b, W2b, b2b, Wp, bp):
  src = edge_index[0]
  dst = edge_index[1]
  e = src.shape[0]
  nchunks = -(-e // (NW * CHUNK))
  e_pad = NW * CHUNK * nchunks
  src3 = jnp.concatenate(
      [src, jnp.zeros((e_pad - e,), jnp.int32)]).reshape(NW, nchunks, CHUNK)
  dst3 = jnp.concatenate(
      [dst, jnp.full((e_pad - e,), TRASH, jnp.int32)]).reshape(
          NW, nchunks, CHUNK)
  zrows = jnp.zeros((ROWS_PER_TILE, D), jnp.float32)

  agg_a = _sc_aggregate(x, src3, dst3, zrows)
  h1 = _tc_mlp(x, agg_a[0, :N], agg_a[1, :N], W1a, b1a, W2a, b2a)
  agg_b = _sc_aggregate(h1, src3, dst3, zrows)
  return _tc_mlp_head(h1, agg_b[0, :N], agg_b[1, :N], W1b, b1b, W2b, b2b,
                      Wp, bp)
